# baseline (speedup 1.0000x reference)
import jax, jax.numpy as jnp

N = 10000
NGRAPHS = 64

def _gcn(x, ei, W, b):
    loop = jnp.arange(N, dtype=ei.dtype)
    src = jnp.concatenate([ei[0], loop])
    dst = jnp.concatenate([ei[1], loop])
    ew = jnp.ones((src.shape[0],), dtype=jnp.float32)
    deg = jax.ops.segment_sum(ew, dst, num_segments=N)
    dinv = jnp.where(deg > 0, jax.lax.rsqrt(jnp.maximum(deg, 1e-12)), 0.0)
    norm = dinv[src] * dinv[dst]
    h = x @ W
    msg = h[src] * norm[:, None]
    out = jax.ops.segment_sum(msg, dst, num_segments=N)
    return out + b

def _pool(x, batch):
    sums = jax.ops.segment_sum(x, batch, num_segments=NGRAPHS)
    cnt = jax.ops.segment_sum(jnp.ones((x.shape[0],), jnp.float32), batch, num_segments=NGRAPHS)
    return sums / jnp.clip(cnt, 1.0)[:, None]

def kernel(x0, x1, edge_index0, edge_index1, batch0, batch1,
           W1_0, b1_0, W1_1, b1_1, Wc0, bc0, Wc1, bc1, Wl1, bl1, Wl2, bl2):
    ys = []
    for (x, ei, batch, W1, b1) in ((x0, edge_index0, batch0, W1_0, b1_0),
                                   (x1, edge_index1, batch1, W1_1, b1_1)):
        h = jax.nn.relu(_gcn(x, ei, W1, b1))
        h = jax.nn.relu(_gcn(h, ei, Wc0, bc0))
        h = jax.nn.relu(_gcn(h, ei, Wc1, bc1))
        g = _pool(h, batch)
        y = jax.nn.relu(g @ Wl1 + bl1)
        y = y @ Wl2 + bl2
        ys.append(y)
    return jnp.stack(ys)


# speedup vs baseline: 3.0211x; 3.0211x over previous
import jax
import jax.numpy as jnp
from jax import lax
from jax.experimental import pallas as pl
from jax.experimental.pallas import tpu as pltpu
from jax.experimental.pallas import tpu_sc as plsc

N = 10000
E = 160000
DIN = 256
HID = 512
NCLS = 10
NGRAPHS = 64

NCORES = 2
NSUB = 16
NTILES = NCORES * NSUB
EPT = E // NTILES
EPT_PAD = 5008
NROWS128 = 40
NBUCKET = 16
NPAD = 10240
BSPAN = NPAD // NBUCKET
SUBROWS = 160
NSUBR = NPAD // SUBROWS
CAP = 6144
STRIP = 2048
GB = 32
RB = 1024
_PREC = lax.Precision.HIGHEST

import functools


@functools.lru_cache(maxsize=None)
def _mesh():
    return plsc.VectorSubcoreMesh(core_axis_name="c", subcore_axis_name="s",
                                  num_cores=NCORES, num_subcores=NSUB)


def _iota16():
    return lax.iota(jnp.int32, 16)


def _partition_body(src_hbm, dst_hbm,
                    bsrc_hbm, bdst_hbm, cnt16_hbm, csc_hbm,
                    sv, dv, dv2, bs, bd, cw, ones128, z640, csc_sp):
    c = lax.axis_index("c")
    s = lax.axis_index("s")
    tid = c * NSUB + s
    ebase = tid * EPT

    pltpu.sync_copy(src_hbm.at[pl.ds(ebase, EPT)], sv.at[pl.ds(0, EPT)])
    pltpu.sync_copy(dst_hbm.at[pl.ds(ebase, EPT)], dv.at[pl.ds(0, EPT)])
    tail = dv[pl.ds(EPT_PAD - 16, 16)]
    dv[pl.ds(EPT_PAD - 16, 16)] = jnp.where(_iota16() < 8, tail, N + _iota16())

    def _zfill(i, _):
        z640[pl.ds(i * 16, 16)] = jnp.zeros((16,), jnp.int32)
        return 0
    lax.fori_loop(0, 40, _zfill, 0)
    pltpu.sync_copy(z640, csc_sp.at[pl.ds(s * 640, 640)])

    for q in range(8):
        ones128[pl.ds(q * 16, 16)] = jnp.ones((16,), jnp.int32)

    for j in range(NROWS128 - 1):
        pltpu.sync_copy(dst_hbm.at[pl.ds(ebase + j * 128, 128)], dv2.at[j])
    r39 = dv2.at[NROWS128 - 1]
    r39[pl.ds(0, 16)] = dv[pl.ds(EPT_PAD - 16, 16)]
    for q in range(1, 8):
        r39[pl.ds(q * 16, 16)] = N + _iota16()

    plsc.subcore_barrier()

    def _hist(j, _):
        pltpu.sync_copy(ones128, csc_sp.at[dv2.at[j]], add=True)
        return 0
    lax.fori_loop(0, NROWS128, _hist, 0)

    plsc.subcore_barrier()
    pltpu.sync_copy(csc_sp.at[pl.ds(s * 640, 640)],
                    csc_hbm.at[pl.ds(c * NPAD + s * 640, 640)])

    ptrs = []
    for k in range(NBUCKET):
        lo = k * BSPAN
        hi = min(lo + BSPAN, N)

        def _compact(i, ptr, lo=lo, hi=hi):
            dvv = dv[pl.ds(i * 16, 16)]
            svv = sv[pl.ds(i * 16, 16)]
            m = (dvv >= lo) & (dvv < hi)
            mi = m.astype(jnp.int32)
            excl = plsc.cumsum(mi) - mi
            pos = jnp.where(m, ptr + excl, CAP + _iota16())
            plsc.store_scatter(bd, [pos], dvv - lo)
            plsc.store_scatter(bs, [pos], svv)
            return ptr + jnp.sum(mi)

        ptrs.append(lax.fori_loop(0, EPT_PAD // 16, _compact, jnp.int32(0)))
        pltpu.sync_copy(bs.at[pl.ds(0, CAP)],
                        bsrc_hbm.at[pl.ds((tid * NBUCKET + k) * CAP, CAP)])
        pltpu.sync_copy(bd.at[pl.ds(0, CAP)],
                        bdst_hbm.at[pl.ds((tid * NBUCKET + k) * CAP, CAP)])

    cvec = jnp.zeros((16,), jnp.int32)
    for k in range(NBUCKET):
        cvec = jnp.where(_iota16() == k, jnp.full((16,), ptrs[k]), cvec)
    cw[pl.ds(0, 16)] = cvec
    pltpu.sync_copy(cw, cnt16_hbm.at[pl.ds(tid * 16, 16)])


@functools.lru_cache(maxsize=None)
def _partition_call():
    return pl.kernel(
    _partition_body,
    out_type=(
        jax.ShapeDtypeStruct((NTILES * NBUCKET * CAP,), jnp.int32),
        jax.ShapeDtypeStruct((NTILES * NBUCKET * CAP,), jnp.int32),
        jax.ShapeDtypeStruct((NTILES * 16,), jnp.int32),
        jax.ShapeDtypeStruct((NCORES * NPAD,), jnp.int32),
    ),
    mesh=_mesh(),
    scratch_types=(
        pltpu.VMEM((EPT_PAD,), jnp.int32),
        pltpu.VMEM((EPT_PAD,), jnp.int32),
        pltpu.VMEM((NROWS128, 128), jnp.int32),
        pltpu.VMEM((CAP + 16,), jnp.int32),
        pltpu.VMEM((CAP + 16,), jnp.int32),
        pltpu.VMEM((16,), jnp.int32),
        pltpu.VMEM((128,), jnp.int32),
        pltpu.VMEM((640,), jnp.int32),
        pltpu.VMEM_SHARED((NPAD,), jnp.int32),
    ),
    name="gcn_edge_partition",
    compiler_params=pltpu.CompilerParams(needs_layout_passes=False),
    )


def _agg_body(u_hbm, bsrc_hbm, bdst_hbm, cnt16_hbm, z_hbm,
              s_out_hbm,
              cntv, dbuf, sbuf, lsrc, ldst, gidx, rows0, rows1, acc,
              sem0, sem1):
    c = lax.axis_index("c")
    s = lax.axis_index("s")
    tid = c * NSUB + s

    pltpu.sync_copy(cnt16_hbm, cntv.at[pl.ds(0, NTILES * 16)])

    for j in range(NSUBR // NTILES):
        r = j * NTILES + tid
        b = r // (BSPAN // SUBROWS)
        sublo = (r % (BSPAN // SUBROWS)) * SUBROWS

        pltpu.sync_copy(z_hbm, acc)

        def _per_tile(t, _, b=b, sublo=sublo):
            cnt = cntv[pl.ds(t * 16 + b, 16)][0]
            base = (t * NBUCKET + b) * CAP

            def _per_strip(g, _, cnt=cnt, base=base, sublo=sublo):
                pltpu.sync_copy(
                    bdst_hbm.at[pl.ds(base + g * STRIP, STRIP)], dbuf)
                pltpu.sync_copy(
                    bsrc_hbm.at[pl.ds(base + g * STRIP, STRIP)], sbuf)

                def _scan(v, lptr, g=g, cnt=cnt, sublo=sublo):
                    dl = dbuf[pl.ds(v * 16, 16)]
                    sv = sbuf[pl.ds(v * 16, 16)]
                    pos = g * STRIP + v * 16 + _iota16()
                    m = (dl >= sublo) & (dl < sublo + SUBROWS) & (pos < cnt)
                    mi = m.astype(jnp.int32)
                    excl = plsc.cumsum(mi) - mi
                    posi = jnp.where(m, lptr + excl, STRIP + _iota16())
                    plsc.store_scatter(lsrc, [posi], sv)
                    plsc.store_scatter(ldst, [posi], dl - sublo)
                    return lptr + jnp.sum(mi)

                lptr = lax.fori_loop(0, STRIP // 16, _scan, jnp.int32(0))
                nb = (lptr + GB - 1) // GB

                def _fill(gref, bi, lptr=lptr):
                    for q in range(GB // 16):
                        pos = bi * GB + q * 16 + _iota16()
                        valid = pos < lptr
                        sv = lsrc[pl.ds(bi * GB + q * 16, 16)]
                        gref[pl.ds(q * 16, 16)] = jnp.where(
                            valid, sv, q * 16 + _iota16())

                def _issue(bi):
                    par = lax.rem(bi, 2)

                    @pl.when(par == 0)
                    def _():
                        _fill(gidx.at[0], bi)
                        pltpu.make_async_copy(
                            u_hbm.at[gidx.at[0]], rows0, sem0).start()

                    @pl.when(par == 1)
                    def _():
                        _fill(gidx.at[1], bi)
                        pltpu.make_async_copy(
                            u_hbm.at[gidx.at[1]], rows1, sem1).start()

                def _drain_acc(rows_r, bi, lptr=lptr):
                    mcount = jnp.minimum(GB, lptr - bi * GB)

                    def _acc_e(e, _):
                        dl = ldst[pl.ds(bi * GB + e, 16)][0]
                        for cb in range(HID // 16):
                            plsc.addupdate(
                                acc.at[dl, pl.ds(cb * 16, 16)],
                                rows_r[e, pl.ds(cb * 16, 16)])
                        return 0

                    lax.fori_loop(0, mcount, _acc_e, 0)

                def _gbatch(bi, _, nb=nb):
                    @pl.when(bi + 1 < nb)
                    def _():
                        _issue(bi + 1)

                    par = lax.rem(bi, 2)

                    @pl.when(par == 0)
                    def _():
                        pltpu.make_async_copy(
                            u_hbm.at[gidx.at[0]], rows0, sem0).wait()
                        _drain_acc(rows0, bi)

                    @pl.when(par == 1)
                    def _():
                        pltpu.make_async_copy(
                            u_hbm.at[gidx.at[1]], rows1, sem1).wait()
                        _drain_acc(rows1, bi)

                    return 0

                @pl.when(nb > 0)
                def _():
                    _issue(jnp.int32(0))

                lax.fori_loop(0, nb, _gbatch, 0)
                return 0

            nst = (cnt + STRIP - 1) // STRIP
            lax.fori_loop(0, nst, _per_strip, 0)
            return 0

        lax.fori_loop(0, NTILES, _per_tile, 0)

        pltpu.sync_copy(acc, s_out_hbm.at[pl.ds(r * SUBROWS, SUBROWS)])


@functools.lru_cache(maxsize=None)
def _agg_call():
    return pl.kernel(
    _agg_body,
    out_type=jax.ShapeDtypeStruct((NPAD, HID), jnp.float32),
    mesh=_mesh(),
    scratch_types=(
        pltpu.VMEM((NTILES * 16 + 16,), jnp.int32),
        pltpu.VMEM((STRIP,), jnp.int32),
        pltpu.VMEM((STRIP,), jnp.int32),
        pltpu.VMEM((STRIP + 16,), jnp.int32),
        pltpu.VMEM((STRIP + 16,), jnp.int32),
        pltpu.VMEM((2, GB), jnp.int32),
        pltpu.VMEM((GB, HID), jnp.float32),
        pltpu.VMEM((GB, HID), jnp.float32),
        pltpu.VMEM((SUBROWS, HID), jnp.float32),
        pltpu.SemaphoreType.DMA,
        pltpu.SemaphoreType.DMA,
    ),
    name="gcn_edge_aggregate",
    compiler_params=pltpu.CompilerParams(needs_layout_passes=False),
    )


def _mm1_body(x_ref, w_ref, c0_ref, c1_ref, u_ref, dinv_ref):
    deg = (c0_ref[...] + c1_ref[...] + 1).astype(jnp.float32)
    dinv = lax.rsqrt(deg)
    h = jnp.dot(x_ref[...], w_ref[...], preferred_element_type=jnp.float32,
                precision=_PREC)
    u_ref[...] = h * dinv
    dinv_ref[...] = dinv


def _mm1(x, w, c0, c1):
    k = x.shape[1]
    return pl.pallas_call(
        _mm1_body,
        grid=(pl.cdiv(N, RB),),
        in_specs=[
            pl.BlockSpec((RB, k), lambda i: (i, 0)),
            pl.BlockSpec((k, HID), lambda i: (0, 0)),
            pl.BlockSpec((RB, 1), lambda i: (i, 0)),
            pl.BlockSpec((RB, 1), lambda i: (i, 0)),
        ],
        out_specs=[
            pl.BlockSpec((RB, HID), lambda i: (i, 0)),
            pl.BlockSpec((RB, 1), lambda i: (i, 0)),
        ],
        out_shape=[
            jax.ShapeDtypeStruct((N, HID), jnp.float32),
            jax.ShapeDtypeStruct((N, 1), jnp.float32),
        ],
    )(x, w, c0, c1)


def _fused_body(s_ref, u_ref, dinv_ref, b_ref, w_ref, out_ref):
    dinv = dinv_ref[...]
    a = jnp.maximum((s_ref[...] + u_ref[...]) * dinv + b_ref[...], 0.0)
    out_ref[...] = jnp.dot(a, w_ref[...], preferred_element_type=jnp.float32,
                           precision=_PREC) * dinv


def _fused_mm(s_agg, u, dinv, b, w):
    return pl.pallas_call(
        _fused_body,
        grid=(pl.cdiv(N, RB),),
        in_specs=[
            pl.BlockSpec((RB, HID), lambda i: (i, 0)),
            pl.BlockSpec((RB, HID), lambda i: (i, 0)),
            pl.BlockSpec((RB, 1), lambda i: (i, 0)),
            pl.BlockSpec((1, HID), lambda i: (0, 0)),
            pl.BlockSpec((HID, HID), lambda i: (0, 0)),
        ],
        out_specs=pl.BlockSpec((RB, HID), lambda i: (i, 0)),
        out_shape=jax.ShapeDtypeStruct((N, HID), jnp.float32),
    )(s_agg, u, dinv, b, w)


def _pool_body(s_ref, u_ref, dinv_ref, b_ref, batch_ref,
               wl1_ref, bl1_ref, wl2_ref, bl2_ref, y_ref, gsum, csum):
    i = pl.program_id(0)

    @pl.when(i == 0)
    def _():
        gsum[...] = jnp.zeros_like(gsum)
        csum[...] = jnp.zeros_like(csum)

    dinv = dinv_ref[...]
    a = jnp.maximum((s_ref[...] + u_ref[...]) * dinv + b_ref[...], 0.0)
    rowpos = lax.broadcasted_iota(jnp.int32, (1, RB), 1) + i * RB
    validrow = rowpos < N
    gid = lax.broadcasted_iota(jnp.int32, (NGRAPHS, RB), 0)
    oh = jnp.where((gid == batch_ref[...]) & validrow, 1.0, 0.0)
    gsum[...] += jnp.dot(oh, a, preferred_element_type=jnp.float32,
                         precision=_PREC)
    csum[...] += jnp.sum(oh, axis=1, keepdims=True)

    @pl.when(i == pl.num_programs(0) - 1)
    def _():
        g = gsum[...] / jnp.maximum(csum[...], 1.0)
        t = jnp.maximum(
            jnp.dot(g, wl1_ref[...], preferred_element_type=jnp.float32,
                    precision=_PREC) + bl1_ref[...], 0.0)
        y_ref[...] = jnp.dot(t, wl2_ref[...],
                             preferred_element_type=jnp.float32,
                             precision=_PREC) + bl2_ref[...]


def _pool_head(s_agg, u, dinv, b, batch2d, wl1, bl1, wl2, bl2):
    return pl.pallas_call(
        _pool_body,
        grid=(pl.cdiv(N, RB),),
        in_specs=[
            pl.BlockSpec((RB, HID), lambda i: (i, 0)),
            pl.BlockSpec((RB, HID), lambda i: (i, 0)),
            pl.BlockSpec((RB, 1), lambda i: (i, 0)),
            pl.BlockSpec((1, HID), lambda i: (0, 0)),
            pl.BlockSpec((1, RB), lambda i: (0, i)),
            pl.BlockSpec((HID, HID), lambda i: (0, 0)),
            pl.BlockSpec((1, HID), lambda i: (0, 0)),
            pl.BlockSpec((HID, NCLS), lambda i: (0, 0)),
            pl.BlockSpec((1, NCLS), lambda i: (0, 0)),
        ],
        out_specs=pl.BlockSpec((NGRAPHS, NCLS), lambda i: (0, 0)),
        out_shape=jax.ShapeDtypeStruct((NGRAPHS, NCLS), jnp.float32),
        scratch_shapes=[
            pltpu.VMEM((NGRAPHS, HID), jnp.float32),
            pltpu.VMEM((NGRAPHS, 1), jnp.float32),
        ],
    )(s_agg, u, dinv, b, batch2d, wl1, bl1, wl2, bl2)


def kernel(x0, x1, edge_index0, edge_index1, batch0, batch1,
           W1_0, b1_0, W1_1, b1_1, Wc0, bc0, Wc1, bc1,
           Wl1, bl1, Wl2, bl2):
    zrows = jnp.zeros((SUBROWS, HID), jnp.float32)
    ys = []
    for (x, ei, batch, w1, b1) in ((x0, edge_index0, batch0, W1_0, b1_0),
                                   (x1, edge_index1, batch1, W1_1, b1_1)):
        src = ei[0]
        dst = ei[1]
        bsrc, bdst, cnt16, csc = _partition_call()(src, dst)
        c0 = csc[:NPAD].reshape(NPAD, 1)
        c1 = csc[NPAD:].reshape(NPAD, 1)

        u1, dinv = _mm1(x, w1, c0, c1)
        s1 = _agg_call()(u1, bsrc, bdst, cnt16, zrows)
        u2 = _fused_mm(s1, u1, dinv, b1.reshape(1, HID), Wc0)
        s2 = _agg_call()(u2, bsrc, bdst, cnt16, zrows)
        u3 = _fused_mm(s2, u2, dinv, bc0.reshape(1, HID), Wc1)
        s3 = _agg_call()(u3, bsrc, bdst, cnt16, zrows)
        y = _pool_head(s3, u3, dinv, bc1.reshape(1, HID),
                       batch.reshape(1, N), Wl1, bl1.reshape(1, HID),
                       Wl2, bl2.reshape(1, NCLS))
        ys.append(y)
    return jnp.stack(ys)


# speedup vs baseline: 3.1079x; 1.0287x over previous
import jax
import jax.numpy as jnp
from jax import lax
from jax.experimental import pallas as pl
from jax.experimental.pallas import tpu as pltpu
from jax.experimental.pallas import tpu_sc as plsc

N = 10000
E = 160000
DIN = 256
HID = 512
NCLS = 10
NGRAPHS = 64

NCORES = 2
NSUB = 16
NTILES = NCORES * NSUB
EPT = E // NTILES
EPT_PAD = 5008
NROWS128 = 40
NBUCKET = 16
NPAD = 10240
BSPAN = NPAD // NBUCKET
SUBROWS = 160
NSUBR = NPAD // SUBROWS
CAP = 6144
STRIP = 2048
GB = 32
RB = 1024
_PREC = lax.Precision.HIGHEST

import functools


@functools.lru_cache(maxsize=None)
def _mesh():
    return plsc.VectorSubcoreMesh(core_axis_name="c", subcore_axis_name="s",
                                  num_cores=NCORES, num_subcores=NSUB)


def _iota16():
    return lax.iota(jnp.int32, 16)


def _partition_body(src_hbm, dst_hbm,
                    bsrc_hbm, bdst_hbm, cnt16_hbm, csc_hbm,
                    sv, dv, dv2, bs, bd, cw, ones128, z640, csc_sp):
    c = lax.axis_index("c")
    s = lax.axis_index("s")
    tid = c * NSUB + s
    ebase = tid * EPT

    pltpu.sync_copy(src_hbm.at[pl.ds(ebase, EPT)], sv.at[pl.ds(0, EPT)])
    pltpu.sync_copy(dst_hbm.at[pl.ds(ebase, EPT)], dv.at[pl.ds(0, EPT)])
    tail = dv[pl.ds(EPT_PAD - 16, 16)]
    dv[pl.ds(EPT_PAD - 16, 16)] = jnp.where(_iota16() < 8, tail, N + _iota16())

    def _zfill(i, _):
        z640[pl.ds(i * 16, 16)] = jnp.zeros((16,), jnp.int32)
        return 0
    lax.fori_loop(0, 40, _zfill, 0)
    pltpu.sync_copy(z640, csc_sp.at[pl.ds(s * 640, 640)])

    for q in range(8):
        ones128[pl.ds(q * 16, 16)] = jnp.ones((16,), jnp.int32)

    for j in range(NROWS128 - 1):
        pltpu.sync_copy(dst_hbm.at[pl.ds(ebase + j * 128, 128)], dv2.at[j])
    r39 = dv2.at[NROWS128 - 1]
    r39[pl.ds(0, 16)] = dv[pl.ds(EPT_PAD - 16, 16)]
    for q in range(1, 8):
        r39[pl.ds(q * 16, 16)] = N + _iota16()

    plsc.subcore_barrier()

    def _hist(j, _):
        pltpu.sync_copy(ones128, csc_sp.at[dv2.at[j]], add=True)
        return 0
    lax.fori_loop(0, NROWS128, _hist, 0)

    plsc.subcore_barrier()
    pltpu.sync_copy(csc_sp.at[pl.ds(s * 640, 640)],
                    csc_hbm.at[pl.ds(c * NPAD + s * 640, 640)])

    ptrs = []
    for k in range(NBUCKET):
        lo = k * BSPAN
        hi = min(lo + BSPAN, N)

        def _compact(i, ptr, lo=lo, hi=hi):
            dvv = dv[pl.ds(i * 16, 16)]
            svv = sv[pl.ds(i * 16, 16)]
            m = (dvv >= lo) & (dvv < hi)
            mi = m.astype(jnp.int32)
            excl = plsc.cumsum(mi) - mi
            pos = jnp.where(m, ptr + excl, CAP + _iota16())
            plsc.store_scatter(bd, [pos], dvv - lo)
            plsc.store_scatter(bs, [pos], svv)
            return ptr + jnp.sum(mi)

        ptrs.append(lax.fori_loop(0, EPT_PAD // 16, _compact, jnp.int32(0)))
        pltpu.sync_copy(bs.at[pl.ds(0, CAP)],
                        bsrc_hbm.at[pl.ds((tid * NBUCKET + k) * CAP, CAP)])
        pltpu.sync_copy(bd.at[pl.ds(0, CAP)],
                        bdst_hbm.at[pl.ds((tid * NBUCKET + k) * CAP, CAP)])

    cvec = jnp.zeros((16,), jnp.int32)
    for k in range(NBUCKET):
        cvec = jnp.where(_iota16() == k, jnp.full((16,), ptrs[k]), cvec)
    cw[pl.ds(0, 16)] = cvec
    pltpu.sync_copy(cw, cnt16_hbm.at[pl.ds(tid * 16, 16)])


@functools.lru_cache(maxsize=None)
def _partition_call():
    return pl.kernel(
    _partition_body,
    out_type=(
        jax.ShapeDtypeStruct((NTILES * NBUCKET * CAP,), jnp.int32),
        jax.ShapeDtypeStruct((NTILES * NBUCKET * CAP,), jnp.int32),
        jax.ShapeDtypeStruct((NTILES * 16,), jnp.int32),
        jax.ShapeDtypeStruct((NCORES * NPAD,), jnp.int32),
    ),
    mesh=_mesh(),
    scratch_types=(
        pltpu.VMEM((EPT_PAD,), jnp.int32),
        pltpu.VMEM((EPT_PAD,), jnp.int32),
        pltpu.VMEM((NROWS128, 128), jnp.int32),
        pltpu.VMEM((CAP + 16,), jnp.int32),
        pltpu.VMEM((CAP + 16,), jnp.int32),
        pltpu.VMEM((16,), jnp.int32),
        pltpu.VMEM((128,), jnp.int32),
        pltpu.VMEM((640,), jnp.int32),
        pltpu.VMEM_SHARED((NPAD,), jnp.int32),
    ),
    name="gcn_edge_partition",
    compiler_params=pltpu.CompilerParams(needs_layout_passes=False),
    )


def _agg_body(u_hbm, bsrc_hbm, bdst_hbm, cnt16_hbm, z_hbm,
              s_out_hbm,
              cntv, dbuf0, dbuf1, sbuf0, sbuf1, lsrc, ldst, gidx, rows0, rows1, acc,
              sem0, sem1, lsem0, lsem1):
    c = lax.axis_index("c")
    s = lax.axis_index("s")
    tid = c * NSUB + s

    pltpu.sync_copy(cnt16_hbm, cntv.at[pl.ds(0, NTILES * 16)])

    for j in range(NSUBR // NTILES):
        r = j * NTILES + tid
        b = r // (BSPAN // SUBROWS)
        sublo = (r % (BSPAN // SUBROWS)) * SUBROWS

        pltpu.sync_copy(z_hbm, acc)

        nstrip = CAP // STRIP

        def _slot_issue(i, par, b=b):
            t = i // nstrip
            g = lax.rem(i, nstrip)
            cnt_t = cntv[pl.ds(t * 16 + b, 16)][0]

            @pl.when(g * STRIP < cnt_t)
            def _():
                base = (t * NBUCKET + b) * CAP + g * STRIP

                @pl.when(par == 0)
                def _():
                    pltpu.make_async_copy(
                        bdst_hbm.at[pl.ds(base, STRIP)], dbuf0,
                        lsem0).start()
                    pltpu.make_async_copy(
                        bsrc_hbm.at[pl.ds(base, STRIP)], sbuf0,
                        lsem0).start()

                @pl.when(par == 1)
                def _():
                    pltpu.make_async_copy(
                        bdst_hbm.at[pl.ds(base, STRIP)], dbuf1,
                        lsem1).start()
                    pltpu.make_async_copy(
                        bsrc_hbm.at[pl.ds(base, STRIP)], sbuf1,
                        lsem1).start()

        def _per_slot(i, _, b=b, sublo=sublo):
            par = lax.rem(i, 2)

            @pl.when(i + 1 < NTILES * nstrip)
            def _():
                _slot_issue(i + 1, 1 - par)

            g = lax.rem(i, nstrip)
            cnt = cntv[pl.ds((i // nstrip) * 16 + b, 16)][0]

            @pl.when(g * STRIP < cnt)
            def _():
                @pl.when(par == 0)
                def _():
                    pltpu.make_async_copy(
                        bdst_hbm.at[pl.ds(0, STRIP)], dbuf0,
                        lsem0).wait()
                    pltpu.make_async_copy(
                        bsrc_hbm.at[pl.ds(0, STRIP)], sbuf0,
                        lsem0).wait()
                    _strip_work(dbuf0, sbuf0, g, cnt, sublo)

                @pl.when(par == 1)
                def _():
                    pltpu.make_async_copy(
                        bdst_hbm.at[pl.ds(0, STRIP)], dbuf1,
                        lsem1).wait()
                    pltpu.make_async_copy(
                        bsrc_hbm.at[pl.ds(0, STRIP)], sbuf1,
                        lsem1).wait()
                    _strip_work(dbuf1, sbuf1, g, cnt, sublo)

            return 0

        def _strip_work(drow, srow, g, cnt, sublo):

                def _scan(v, lptr, g=g, cnt=cnt, sublo=sublo):
                    dl = drow[pl.ds(v * 16, 16)]
                    sv = srow[pl.ds(v * 16, 16)]
                    pos = g * STRIP + v * 16 + _iota16()
                    m = (dl >= sublo) & (dl < sublo + SUBROWS) & (pos < cnt)
                    mi = m.astype(jnp.int32)
                    excl = plsc.cumsum(mi) - mi
                    posi = jnp.where(m, lptr + excl, STRIP + _iota16())
                    plsc.store_scatter(lsrc, [posi], sv)
                    plsc.store_scatter(ldst, [posi], dl - sublo)
                    return lptr + jnp.sum(mi)

                lptr = lax.fori_loop(0, STRIP // 16, _scan, jnp.int32(0))
                nb = (lptr + GB - 1) // GB

                def _fill(gref, bi, lptr=lptr):
                    for q in range(GB // 16):
                        pos = bi * GB + q * 16 + _iota16()
                        valid = pos < lptr
                        sv = lsrc[pl.ds(bi * GB + q * 16, 16)]
                        gref[pl.ds(q * 16, 16)] = jnp.where(
                            valid, sv, q * 16 + _iota16())

                def _issue(bi):
                    par = lax.rem(bi, 2)

                    @pl.when(par == 0)
                    def _():
                        _fill(gidx.at[0], bi)
                        pltpu.make_async_copy(
                            u_hbm.at[gidx.at[0]], rows0, sem0).start()

                    @pl.when(par == 1)
                    def _():
                        _fill(gidx.at[1], bi)
                        pltpu.make_async_copy(
                            u_hbm.at[gidx.at[1]], rows1, sem1).start()

                def _drain_acc(rows_r, bi, lptr=lptr):
                    mcount = jnp.minimum(GB, lptr - bi * GB)

                    def _acc_e(e, _):
                        dl = ldst[pl.ds(bi * GB + e, 16)][0]
                        for cb in range(HID // 16):
                            plsc.addupdate(
                                acc.at[dl, pl.ds(cb * 16, 16)],
                                rows_r[e, pl.ds(cb * 16, 16)])
                        return 0

                    lax.fori_loop(0, mcount, _acc_e, 0)

                def _gbatch(bi, _, nb=nb):
                    @pl.when(bi + 1 < nb)
                    def _():
                        _issue(bi + 1)

                    par = lax.rem(bi, 2)

                    @pl.when(par == 0)
                    def _():
                        pltpu.make_async_copy(
                            u_hbm.at[gidx.at[0]], rows0, sem0).wait()
                        _drain_acc(rows0, bi)

                    @pl.when(par == 1)
                    def _():
                        pltpu.make_async_copy(
                            u_hbm.at[gidx.at[1]], rows1, sem1).wait()
                        _drain_acc(rows1, bi)

                    return 0

                @pl.when(nb > 0)
                def _():
                    _issue(jnp.int32(0))

                lax.fori_loop(0, nb, _gbatch, 0)

        _slot_issue(jnp.int32(0), jnp.int32(0))
        lax.fori_loop(0, NTILES * nstrip, _per_slot, 0)

        pltpu.sync_copy(acc, s_out_hbm.at[pl.ds(r * SUBROWS, SUBROWS)])


@functools.lru_cache(maxsize=None)
def _agg_call():
    return pl.kernel(
    _agg_body,
    out_type=jax.ShapeDtypeStruct((NPAD, HID), jnp.float32),
    mesh=_mesh(),
    scratch_types=(
        pltpu.VMEM((NTILES * 16 + 16,), jnp.int32),
        pltpu.VMEM((STRIP,), jnp.int32),
        pltpu.VMEM((STRIP,), jnp.int32),
        pltpu.VMEM((STRIP,), jnp.int32),
        pltpu.VMEM((STRIP,), jnp.int32),
        pltpu.VMEM((STRIP + 16,), jnp.int32),
        pltpu.VMEM((STRIP + 16,), jnp.int32),
        pltpu.VMEM((2, GB), jnp.int32),
        pltpu.VMEM((GB, HID), jnp.float32),
        pltpu.VMEM((GB, HID), jnp.float32),
        pltpu.VMEM((SUBROWS, HID), jnp.float32),
        pltpu.SemaphoreType.DMA,
        pltpu.SemaphoreType.DMA,
        pltpu.SemaphoreType.DMA,
        pltpu.SemaphoreType.DMA,
    ),
    name="gcn_edge_aggregate",
    compiler_params=pltpu.CompilerParams(needs_layout_passes=False),
    )


def _mm1_body(x_ref, w_ref, c0_ref, c1_ref, u_ref, dinv_ref):
    deg = (c0_ref[...] + c1_ref[...] + 1).astype(jnp.float32)
    dinv = lax.rsqrt(deg)
    h = jnp.dot(x_ref[...], w_ref[...], preferred_element_type=jnp.float32,
                precision=_PREC)
    u_ref[...] = h * dinv
    dinv_ref[...] = dinv


def _mm1(x, w, c0, c1):
    k = x.shape[1]
    return pl.pallas_call(
        _mm1_body,
        grid=(pl.cdiv(N, RB),),
        in_specs=[
            pl.BlockSpec((RB, k), lambda i: (i, 0)),
            pl.BlockSpec((k, HID), lambda i: (0, 0)),
            pl.BlockSpec((RB, 1), lambda i: (i, 0)),
            pl.BlockSpec((RB, 1), lambda i: (i, 0)),
        ],
        out_specs=[
            pl.BlockSpec((RB, HID), lambda i: (i, 0)),
            pl.BlockSpec((RB, 1), lambda i: (i, 0)),
        ],
        out_shape=[
            jax.ShapeDtypeStruct((N, HID), jnp.float32),
            jax.ShapeDtypeStruct((N, 1), jnp.float32),
        ],
    )(x, w, c0, c1)


def _fused_body(s_ref, u_ref, dinv_ref, b_ref, w_ref, out_ref):
    dinv = dinv_ref[...]
    a = jnp.maximum((s_ref[...] + u_ref[...]) * dinv + b_ref[...], 0.0)
    out_ref[...] = jnp.dot(a, w_ref[...], preferred_element_type=jnp.float32,
                           precision=_PREC) * dinv


def _fused_mm(s_agg, u, dinv, b, w):
    return pl.pallas_call(
        _fused_body,
        grid=(pl.cdiv(N, RB),),
        in_specs=[
            pl.BlockSpec((RB, HID), lambda i: (i, 0)),
            pl.BlockSpec((RB, HID), lambda i: (i, 0)),
            pl.BlockSpec((RB, 1), lambda i: (i, 0)),
            pl.BlockSpec((1, HID), lambda i: (0, 0)),
            pl.BlockSpec((HID, HID), lambda i: (0, 0)),
        ],
        out_specs=pl.BlockSpec((RB, HID), lambda i: (i, 0)),
        out_shape=jax.ShapeDtypeStruct((N, HID), jnp.float32),
    )(s_agg, u, dinv, b, w)


def _pool_body(s_ref, u_ref, dinv_ref, b_ref, batch_ref,
               wl1_ref, bl1_ref, wl2_ref, bl2_ref, y_ref, gsum, csum):
    i = pl.program_id(0)

    @pl.when(i == 0)
    def _():
        gsum[...] = jnp.zeros_like(gsum)
        csum[...] = jnp.zeros_like(csum)

    dinv = dinv_ref[...]
    a = jnp.maximum((s_ref[...] + u_ref[...]) * dinv + b_ref[...], 0.0)
    rowpos = lax.broadcasted_iota(jnp.int32, (1, RB), 1) + i * RB
    validrow = rowpos < N
    gid = lax.broadcasted_iota(jnp.int32, (NGRAPHS, RB), 0)
    oh = jnp.where((gid == batch_ref[...]) & validrow, 1.0, 0.0)
    gsum[...] += jnp.dot(oh, a, preferred_element_type=jnp.float32,
                         precision=_PREC)
    csum[...] += jnp.sum(oh, axis=1, keepdims=True)

    @pl.when(i == pl.num_programs(0) - 1)
    def _():
        g = gsum[...] / jnp.maximum(csum[...], 1.0)
        t = jnp.maximum(
            jnp.dot(g, wl1_ref[...], preferred_element_type=jnp.float32,
                    precision=_PREC) + bl1_ref[...], 0.0)
        y_ref[...] = jnp.dot(t, wl2_ref[...],
                             preferred_element_type=jnp.float32,
                             precision=_PREC) + bl2_ref[...]


def _pool_head(s_agg, u, dinv, b, batch2d, wl1, bl1, wl2, bl2):
    return pl.pallas_call(
        _pool_body,
        grid=(pl.cdiv(N, RB),),
        in_specs=[
            pl.BlockSpec((RB, HID), lambda i: (i, 0)),
            pl.BlockSpec((RB, HID), lambda i: (i, 0)),
            pl.BlockSpec((RB, 1), lambda i: (i, 0)),
            pl.BlockSpec((1, HID), lambda i: (0, 0)),
            pl.BlockSpec((1, RB), lambda i: (0, i)),
            pl.BlockSpec((HID, HID), lambda i: (0, 0)),
            pl.BlockSpec((1, HID), lambda i: (0, 0)),
            pl.BlockSpec((HID, NCLS), lambda i: (0, 0)),
            pl.BlockSpec((1, NCLS), lambda i: (0, 0)),
        ],
        out_specs=pl.BlockSpec((NGRAPHS, NCLS), lambda i: (0, 0)),
        out_shape=jax.ShapeDtypeStruct((NGRAPHS, NCLS), jnp.float32),
        scratch_shapes=[
            pltpu.VMEM((NGRAPHS, HID), jnp.float32),
            pltpu.VMEM((NGRAPHS, 1), jnp.float32),
        ],
    )(s_agg, u, dinv, b, batch2d, wl1, bl1, wl2, bl2)


def kernel(x0, x1, edge_index0, edge_index1, batch0, batch1,
           W1_0, b1_0, W1_1, b1_1, Wc0, bc0, Wc1, bc1,
           Wl1, bl1, Wl2, bl2):
    zrows = jnp.zeros((SUBROWS, HID), jnp.float32)
    ys = []
    for (x, ei, batch, w1, b1) in ((x0, edge_index0, batch0, W1_0, b1_0),
                                   (x1, edge_index1, batch1, W1_1, b1_1)):
        src = ei[0]
        dst = ei[1]
        bsrc, bdst, cnt16, csc = _partition_call()(src, dst)
        c0 = csc[:NPAD].reshape(NPAD, 1)
        c1 = csc[NPAD:].reshape(NPAD, 1)

        u1, dinv = _mm1(x, w1, c0, c1)
        s1 = _agg_call()(u1, bsrc, bdst, cnt16, zrows)
        u2 = _fused_mm(s1, u1, dinv, b1.reshape(1, HID), Wc0)
        s2 = _agg_call()(u2, bsrc, bdst, cnt16, zrows)
        u3 = _fused_mm(s2, u2, dinv, bc0.reshape(1, HID), Wc1)
        s3 = _agg_call()(u3, bsrc, bdst, cnt16, zrows)
        y = _pool_head(s3, u3, dinv, bc1.reshape(1, HID),
                       batch.reshape(1, N), Wl1, bl1.reshape(1, HID),
                       Wl2, bl2.reshape(1, NCLS))
        ys.append(y)
    return jnp.stack(ys)


# speedup vs baseline: 3.4499x; 1.1100x over previous
import jax
import jax.numpy as jnp
from jax import lax
from jax.experimental import pallas as pl
from jax.experimental.pallas import tpu as pltpu
from jax.experimental.pallas import tpu_sc as plsc

N = 10000
E = 160000
DIN = 256
HID = 512
NCLS = 10
NGRAPHS = 64

NCORES = 2
NSUB = 16
NTILES = NCORES * NSUB
EPT = E // NTILES
EPT_PAD = 5008
NROWS128 = 40
NBUCKET = 16
NPAD = 10240
BSPAN = NPAD // NBUCKET
SUBROWS = 160
NSUBR = NPAD // SUBROWS
CAP = 6144
STRIP = 2048
GB = 32
RB = 1024
_PREC = lax.Precision.HIGHEST

import functools


@functools.lru_cache(maxsize=None)
def _mesh():
    return plsc.VectorSubcoreMesh(core_axis_name="c", subcore_axis_name="s",
                                  num_cores=NCORES, num_subcores=NSUB)


def _iota16():
    return lax.iota(jnp.int32, 16)


def _partition_body(src_hbm, dst_hbm,
                    bsrc_hbm, bdst_hbm, cnt16_hbm, csc_hbm,
                    sv, dv, dv2, bs, bd, cw, ones128, z640, csc_sp):
    c = lax.axis_index("c")
    s = lax.axis_index("s")
    tid = c * NSUB + s
    ebase = tid * EPT

    pltpu.sync_copy(src_hbm.at[pl.ds(ebase, EPT)], sv.at[pl.ds(0, EPT)])
    pltpu.sync_copy(dst_hbm.at[pl.ds(ebase, EPT)], dv.at[pl.ds(0, EPT)])
    tail = dv[pl.ds(EPT_PAD - 16, 16)]
    dv[pl.ds(EPT_PAD - 16, 16)] = jnp.where(_iota16() < 8, tail, N + _iota16())

    def _zfill(i, _):
        z640[pl.ds(i * 16, 16)] = jnp.zeros((16,), jnp.int32)
        return 0
    lax.fori_loop(0, 40, _zfill, 0)
    pltpu.sync_copy(z640, csc_sp.at[pl.ds(s * 640, 640)])

    for q in range(8):
        ones128[pl.ds(q * 16, 16)] = jnp.ones((16,), jnp.int32)

    for j in range(NROWS128 - 1):
        pltpu.sync_copy(dst_hbm.at[pl.ds(ebase + j * 128, 128)], dv2.at[j])
    r39 = dv2.at[NROWS128 - 1]
    r39[pl.ds(0, 16)] = dv[pl.ds(EPT_PAD - 16, 16)]
    for q in range(1, 8):
        r39[pl.ds(q * 16, 16)] = N + _iota16()

    plsc.subcore_barrier()

    def _hist(j, _):
        pltpu.sync_copy(ones128, csc_sp.at[dv2.at[j]], add=True)
        return 0
    lax.fori_loop(0, NROWS128, _hist, 0)

    plsc.subcore_barrier()
    pltpu.sync_copy(csc_sp.at[pl.ds(s * 640, 640)],
                    csc_hbm.at[pl.ds(c * NPAD + s * 640, 640)])

    ptrs = []
    for k in range(NBUCKET):
        lo = k * BSPAN
        hi = min(lo + BSPAN, N)

        def _compact(i, ptr, lo=lo, hi=hi):
            dvv = dv[pl.ds(i * 16, 16)]
            svv = sv[pl.ds(i * 16, 16)]
            m = (dvv >= lo) & (dvv < hi)
            mi = m.astype(jnp.int32)
            excl = plsc.cumsum(mi) - mi
            pos = jnp.where(m, ptr + excl, CAP + _iota16())
            plsc.store_scatter(bd, [pos], dvv - lo)
            plsc.store_scatter(bs, [pos], svv)
            return ptr + jnp.sum(mi)

        ptrs.append(lax.fori_loop(0, EPT_PAD // 16, _compact, jnp.int32(0)))
        pltpu.sync_copy(bs.at[pl.ds(0, CAP)],
                        bsrc_hbm.at[pl.ds((tid * NBUCKET + k) * CAP, CAP)])
        pltpu.sync_copy(bd.at[pl.ds(0, CAP)],
                        bdst_hbm.at[pl.ds((tid * NBUCKET + k) * CAP, CAP)])

    cvec = jnp.zeros((16,), jnp.int32)
    for k in range(NBUCKET):
        cvec = jnp.where(_iota16() == k, jnp.full((16,), ptrs[k]), cvec)
    cw[pl.ds(0, 16)] = cvec
    pltpu.sync_copy(cw, cnt16_hbm.at[pl.ds(tid * 16, 16)])


@functools.lru_cache(maxsize=None)
def _partition_call():
    return pl.kernel(
    _partition_body,
    out_type=(
        jax.ShapeDtypeStruct((NTILES * NBUCKET * CAP,), jnp.int32),
        jax.ShapeDtypeStruct((NTILES * NBUCKET * CAP,), jnp.int32),
        jax.ShapeDtypeStruct((NTILES * 16,), jnp.int32),
        jax.ShapeDtypeStruct((NCORES * NPAD,), jnp.int32),
    ),
    mesh=_mesh(),
    scratch_types=(
        pltpu.VMEM((EPT_PAD,), jnp.int32),
        pltpu.VMEM((EPT_PAD,), jnp.int32),
        pltpu.VMEM((NROWS128, 128), jnp.int32),
        pltpu.VMEM((CAP + 16,), jnp.int32),
        pltpu.VMEM((CAP + 16,), jnp.int32),
        pltpu.VMEM((16,), jnp.int32),
        pltpu.VMEM((128,), jnp.int32),
        pltpu.VMEM((640,), jnp.int32),
        pltpu.VMEM_SHARED((NPAD,), jnp.int32),
    ),
    name="gcn_edge_partition",
    compiler_params=pltpu.CompilerParams(needs_layout_passes=False),
    )


def _agg_body(u_hbm, bsrc_hbm, bdst_hbm, cnt16_hbm, z_hbm,
              s_out_hbm,
              cntv, dbuf0, dbuf1, sbuf0, sbuf1, lsrc, ldst, gidx, rows0, rows1, acc,
              sem0, sem1, lsem0, lsem1):
    c = lax.axis_index("c")
    s = lax.axis_index("s")
    tid = c * NSUB + s

    pltpu.sync_copy(cnt16_hbm, cntv.at[pl.ds(0, NTILES * 16)])

    for j in range(NSUBR // NTILES):
        r = j * NTILES + tid
        b = r // (BSPAN // SUBROWS)
        sublo = (r % (BSPAN // SUBROWS)) * SUBROWS

        pltpu.sync_copy(z_hbm, acc)

        nstrip = CAP // STRIP

        def _slot_issue(i, par, b=b):
            t = i // nstrip
            g = lax.rem(i, nstrip)
            cnt_t = cntv[pl.ds(t * 16 + b, 16)][0]

            @pl.when(g * STRIP < cnt_t)
            def _():
                base = (t * NBUCKET + b) * CAP + g * STRIP

                @pl.when(par == 0)
                def _():
                    pltpu.make_async_copy(
                        bdst_hbm.at[pl.ds(base, STRIP)], dbuf0,
                        lsem0).start()
                    pltpu.make_async_copy(
                        bsrc_hbm.at[pl.ds(base, STRIP)], sbuf0,
                        lsem0).start()

                @pl.when(par == 1)
                def _():
                    pltpu.make_async_copy(
                        bdst_hbm.at[pl.ds(base, STRIP)], dbuf1,
                        lsem1).start()
                    pltpu.make_async_copy(
                        bsrc_hbm.at[pl.ds(base, STRIP)], sbuf1,
                        lsem1).start()

        def _per_slot(i, _, b=b, sublo=sublo):
            par = lax.rem(i, 2)

            @pl.when(i + 1 < NTILES * nstrip)
            def _():
                _slot_issue(i + 1, 1 - par)

            g = lax.rem(i, nstrip)
            cnt = cntv[pl.ds((i // nstrip) * 16 + b, 16)][0]

            @pl.when(g * STRIP < cnt)
            def _():
                @pl.when(par == 0)
                def _():
                    pltpu.make_async_copy(
                        bdst_hbm.at[pl.ds(0, STRIP)], dbuf0,
                        lsem0).wait()
                    pltpu.make_async_copy(
                        bsrc_hbm.at[pl.ds(0, STRIP)], sbuf0,
                        lsem0).wait()
                    _strip_work(dbuf0, sbuf0, g, cnt, sublo)

                @pl.when(par == 1)
                def _():
                    pltpu.make_async_copy(
                        bdst_hbm.at[pl.ds(0, STRIP)], dbuf1,
                        lsem1).wait()
                    pltpu.make_async_copy(
                        bsrc_hbm.at[pl.ds(0, STRIP)], sbuf1,
                        lsem1).wait()
                    _strip_work(dbuf1, sbuf1, g, cnt, sublo)

            return 0

        def _strip_work(drow, srow, g, cnt, sublo):

                def _scan(v, lptr, g=g, cnt=cnt, sublo=sublo):
                    dl = drow[pl.ds(v * 16, 16)]
                    sv = srow[pl.ds(v * 16, 16)]
                    pos = g * STRIP + v * 16 + _iota16()
                    m = (dl >= sublo) & (dl < sublo + SUBROWS) & (pos < cnt)
                    mi = m.astype(jnp.int32)
                    incl = plsc.cumsum(mi)
                    posi = jnp.where(m, lptr + incl - mi, STRIP + _iota16())
                    plsc.store_scatter(lsrc, [posi], sv)
                    plsc.store_scatter(ldst, [posi], dl - sublo)
                    return lptr + incl[15]

                nvec = jnp.clip((cnt - g * STRIP + 15) // 16, 0, STRIP // 16)
                lptr = lax.fori_loop(0, nvec, _scan, jnp.int32(0))
                nb = (lptr + GB - 1) // GB

                def _fill(gref, bi, lptr=lptr):
                    for q in range(GB // 16):
                        pos = bi * GB + q * 16 + _iota16()
                        valid = pos < lptr
                        sv = lsrc[pl.ds(bi * GB + q * 16, 16)]
                        gref[pl.ds(q * 16, 16)] = jnp.where(
                            valid, sv, q * 16 + _iota16())

                def _issue(bi):
                    par = lax.rem(bi, 2)

                    @pl.when(par == 0)
                    def _():
                        _fill(gidx.at[0], bi)
                        pltpu.make_async_copy(
                            u_hbm.at[gidx.at[0]], rows0, sem0).start()

                    @pl.when(par == 1)
                    def _():
                        _fill(gidx.at[1], bi)
                        pltpu.make_async_copy(
                            u_hbm.at[gidx.at[1]], rows1, sem1).start()

                def _drain_acc(rows_r, bi, lptr=lptr):
                    mcount = jnp.minimum(GB, lptr - bi * GB)

                    def _acc_e(e, _):
                        dl = ldst[pl.ds(bi * GB + e, 16)][0]
                        for cb in range(HID // 16):
                            plsc.addupdate(
                                acc.at[dl, pl.ds(cb * 16, 16)],
                                rows_r[e, pl.ds(cb * 16, 16)])
                        return 0

                    lax.fori_loop(0, mcount, _acc_e, 0)

                def _gbatch(bi, _, nb=nb):
                    @pl.when(bi + 1 < nb)
                    def _():
                        _issue(bi + 1)

                    par = lax.rem(bi, 2)

                    @pl.when(par == 0)
                    def _():
                        pltpu.make_async_copy(
                            u_hbm.at[gidx.at[0]], rows0, sem0).wait()
                        _drain_acc(rows0, bi)

                    @pl.when(par == 1)
                    def _():
                        pltpu.make_async_copy(
                            u_hbm.at[gidx.at[1]], rows1, sem1).wait()
                        _drain_acc(rows1, bi)

                    return 0

                @pl.when(nb > 0)
                def _():
                    _issue(jnp.int32(0))

                lax.fori_loop(0, nb, _gbatch, 0)

        _slot_issue(jnp.int32(0), jnp.int32(0))
        lax.fori_loop(0, NTILES * nstrip, _per_slot, 0)

        pltpu.sync_copy(acc, s_out_hbm.at[pl.ds(r * SUBROWS, SUBROWS)])


@functools.lru_cache(maxsize=None)
def _agg_call():
    return pl.kernel(
    _agg_body,
    out_type=jax.ShapeDtypeStruct((NPAD, HID), jnp.float32),
    mesh=_mesh(),
    scratch_types=(
        pltpu.VMEM((NTILES * 16 + 16,), jnp.int32),
        pltpu.VMEM((STRIP,), jnp.int32),
        pltpu.VMEM((STRIP,), jnp.int32),
        pltpu.VMEM((STRIP,), jnp.int32),
        pltpu.VMEM((STRIP,), jnp.int32),
        pltpu.VMEM((STRIP + 16,), jnp.int32),
        pltpu.VMEM((STRIP + 16,), jnp.int32),
        pltpu.VMEM((2, GB), jnp.int32),
        pltpu.VMEM((GB, HID), jnp.float32),
        pltpu.VMEM((GB, HID), jnp.float32),
        pltpu.VMEM((SUBROWS, HID), jnp.float32),
        pltpu.SemaphoreType.DMA,
        pltpu.SemaphoreType.DMA,
        pltpu.SemaphoreType.DMA,
        pltpu.SemaphoreType.DMA,
    ),
    name="gcn_edge_aggregate",
    compiler_params=pltpu.CompilerParams(needs_layout_passes=False),
    )


def _mm1_body(x_ref, w_ref, c0_ref, c1_ref, u_ref, dinv_ref):
    deg = (c0_ref[...] + c1_ref[...] + 1).astype(jnp.float32)
    dinv = lax.rsqrt(deg)
    h = jnp.dot(x_ref[...], w_ref[...], preferred_element_type=jnp.float32,
                precision=_PREC)
    u_ref[...] = h * dinv
    dinv_ref[...] = dinv


def _mm1(x, w, c0, c1):
    k = x.shape[1]
    return pl.pallas_call(
        _mm1_body,
        grid=(pl.cdiv(N, RB),),
        in_specs=[
            pl.BlockSpec((RB, k), lambda i: (i, 0)),
            pl.BlockSpec((k, HID), lambda i: (0, 0)),
            pl.BlockSpec((RB, 1), lambda i: (i, 0)),
            pl.BlockSpec((RB, 1), lambda i: (i, 0)),
        ],
        out_specs=[
            pl.BlockSpec((RB, HID), lambda i: (i, 0)),
            pl.BlockSpec((RB, 1), lambda i: (i, 0)),
        ],
        out_shape=[
            jax.ShapeDtypeStruct((N, HID), jnp.float32),
            jax.ShapeDtypeStruct((N, 1), jnp.float32),
        ],
    )(x, w, c0, c1)


def _fused_body(s_ref, u_ref, dinv_ref, b_ref, w_ref, out_ref):
    dinv = dinv_ref[...]
    a = jnp.maximum((s_ref[...] + u_ref[...]) * dinv + b_ref[...], 0.0)
    out_ref[...] = jnp.dot(a, w_ref[...], preferred_element_type=jnp.float32,
                           precision=_PREC) * dinv


def _fused_mm(s_agg, u, dinv, b, w):
    return pl.pallas_call(
        _fused_body,
        grid=(pl.cdiv(N, RB),),
        in_specs=[
            pl.BlockSpec((RB, HID), lambda i: (i, 0)),
            pl.BlockSpec((RB, HID), lambda i: (i, 0)),
            pl.BlockSpec((RB, 1), lambda i: (i, 0)),
            pl.BlockSpec((1, HID), lambda i: (0, 0)),
            pl.BlockSpec((HID, HID), lambda i: (0, 0)),
        ],
        out_specs=pl.BlockSpec((RB, HID), lambda i: (i, 0)),
        out_shape=jax.ShapeDtypeStruct((N, HID), jnp.float32),
    )(s_agg, u, dinv, b, w)


def _pool_body(s_ref, u_ref, dinv_ref, b_ref, batch_ref,
               wl1_ref, bl1_ref, wl2_ref, bl2_ref, y_ref, gsum, csum):
    i = pl.program_id(0)

    @pl.when(i == 0)
    def _():
        gsum[...] = jnp.zeros_like(gsum)
        csum[...] = jnp.zeros_like(csum)

    dinv = dinv_ref[...]
    a = jnp.maximum((s_ref[...] + u_ref[...]) * dinv + b_ref[...], 0.0)
    rowpos = lax.broadcasted_iota(jnp.int32, (1, RB), 1) + i * RB
    validrow = rowpos < N
    gid = lax.broadcasted_iota(jnp.int32, (NGRAPHS, RB), 0)
    oh = jnp.where((gid == batch_ref[...]) & validrow, 1.0, 0.0)
    gsum[...] += jnp.dot(oh, a, preferred_element_type=jnp.float32,
                         precision=_PREC)
    csum[...] += jnp.sum(oh, axis=1, keepdims=True)

    @pl.when(i == pl.num_programs(0) - 1)
    def _():
        g = gsum[...] / jnp.maximum(csum[...], 1.0)
        t = jnp.maximum(
            jnp.dot(g, wl1_ref[...], preferred_element_type=jnp.float32,
                    precision=_PREC) + bl1_ref[...], 0.0)
        y_ref[...] = jnp.dot(t, wl2_ref[...],
                             preferred_element_type=jnp.float32,
                             precision=_PREC) + bl2_ref[...]


def _pool_head(s_agg, u, dinv, b, batch2d, wl1, bl1, wl2, bl2):
    return pl.pallas_call(
        _pool_body,
        grid=(pl.cdiv(N, RB),),
        in_specs=[
            pl.BlockSpec((RB, HID), lambda i: (i, 0)),
            pl.BlockSpec((RB, HID), lambda i: (i, 0)),
            pl.BlockSpec((RB, 1), lambda i: (i, 0)),
            pl.BlockSpec((1, HID), lambda i: (0, 0)),
            pl.BlockSpec((1, RB), lambda i: (0, i)),
            pl.BlockSpec((HID, HID), lambda i: (0, 0)),
            pl.BlockSpec((1, HID), lambda i: (0, 0)),
            pl.BlockSpec((HID, NCLS), lambda i: (0, 0)),
            pl.BlockSpec((1, NCLS), lambda i: (0, 0)),
        ],
        out_specs=pl.BlockSpec((NGRAPHS, NCLS), lambda i: (0, 0)),
        out_shape=jax.ShapeDtypeStruct((NGRAPHS, NCLS), jnp.float32),
        scratch_shapes=[
            pltpu.VMEM((NGRAPHS, HID), jnp.float32),
            pltpu.VMEM((NGRAPHS, 1), jnp.float32),
        ],
    )(s_agg, u, dinv, b, batch2d, wl1, bl1, wl2, bl2)


def kernel(x0, x1, edge_index0, edge_index1, batch0, batch1,
           W1_0, b1_0, W1_1, b1_1, Wc0, bc0, Wc1, bc1,
           Wl1, bl1, Wl2, bl2):
    zrows = jnp.zeros((SUBROWS, HID), jnp.float32)
    ys = []
    for (x, ei, batch, w1, b1) in ((x0, edge_index0, batch0, W1_0, b1_0),
                                   (x1, edge_index1, batch1, W1_1, b1_1)):
        src = ei[0]
        dst = ei[1]
        bsrc, bdst, cnt16, csc = _partition_call()(src, dst)
        c0 = csc[:NPAD].reshape(NPAD, 1)
        c1 = csc[NPAD:].reshape(NPAD, 1)

        u1, dinv = _mm1(x, w1, c0, c1)
        s1 = _agg_call()(u1, bsrc, bdst, cnt16, zrows)
        u2 = _fused_mm(s1, u1, dinv, b1.reshape(1, HID), Wc0)
        s2 = _agg_call()(u2, bsrc, bdst, cnt16, zrows)
        u3 = _fused_mm(s2, u2, dinv, bc0.reshape(1, HID), Wc1)
        s3 = _agg_call()(u3, bsrc, bdst, cnt16, zrows)
        y = _pool_head(s3, u3, dinv, bc1.reshape(1, HID),
                       batch.reshape(1, N), Wl1, bl1.reshape(1, HID),
                       Wl2, bl2.reshape(1, NCLS))
        ys.append(y)
    return jnp.stack(ys)


# speedup vs baseline: 6.1308x; 1.7771x over previous
import jax
import jax.numpy as jnp
from jax import lax
from jax.experimental import pallas as pl
from jax.experimental.pallas import tpu as pltpu
from jax.experimental.pallas import tpu_sc as plsc

N = 10000
E = 160000
DIN = 256
HID = 512
NCLS = 10
NGRAPHS = 64

NCORES = 2
NSUB = 16
NTILES = NCORES * NSUB
EPT = E // NTILES
EPT_PAD = 5008
NROWS128 = 40
NBUCKET = 16
NPAD = 10240
BSPAN = NPAD // NBUCKET
SUBROWS = 160
NSUBR = NPAD // SUBROWS
CAP = 6144
STRIP = 2048
GB = 32
RB = 1024
_PREC = lax.Precision.HIGHEST

import functools


@functools.lru_cache(maxsize=None)
def _mesh():
    return plsc.VectorSubcoreMesh(core_axis_name="c", subcore_axis_name="s",
                                  num_cores=NCORES, num_subcores=NSUB)


def _iota16():
    return lax.iota(jnp.int32, 16)


def _partition_body(src_hbm, dst_hbm,
                    bsrc_hbm, bdst_hbm, cnt16_hbm, csc_hbm,
                    sv, dv, dv2, bs, bd, cw, ones128, z640, csc_sp):
    c = lax.axis_index("c")
    s = lax.axis_index("s")
    tid = c * NSUB + s
    ebase = tid * EPT

    pltpu.sync_copy(src_hbm.at[pl.ds(ebase, EPT)], sv.at[pl.ds(0, EPT)])
    pltpu.sync_copy(dst_hbm.at[pl.ds(ebase, EPT)], dv.at[pl.ds(0, EPT)])
    tail = dv[pl.ds(EPT_PAD - 16, 16)]
    dv[pl.ds(EPT_PAD - 16, 16)] = jnp.where(_iota16() < 8, tail, N + _iota16())

    def _zfill(i, _):
        z640[pl.ds(i * 16, 16)] = jnp.zeros((16,), jnp.int32)
        return 0
    lax.fori_loop(0, 40, _zfill, 0)
    pltpu.sync_copy(z640, csc_sp.at[pl.ds(s * 640, 640)])

    for q in range(8):
        ones128[pl.ds(q * 16, 16)] = jnp.ones((16,), jnp.int32)

    for j in range(NROWS128 - 1):
        pltpu.sync_copy(dst_hbm.at[pl.ds(ebase + j * 128, 128)], dv2.at[j])
    r39 = dv2.at[NROWS128 - 1]
    r39[pl.ds(0, 16)] = dv[pl.ds(EPT_PAD - 16, 16)]
    for q in range(1, 8):
        r39[pl.ds(q * 16, 16)] = N + _iota16()

    plsc.subcore_barrier()

    def _hist(j, _):
        pltpu.sync_copy(ones128, csc_sp.at[dv2.at[j]], add=True)
        return 0
    lax.fori_loop(0, NROWS128, _hist, 0)

    plsc.subcore_barrier()
    pltpu.sync_copy(csc_sp.at[pl.ds(s * 640, 640)],
                    csc_hbm.at[pl.ds(c * NPAD + s * 640, 640)])

    ptrs = []
    for k in range(NBUCKET):
        lo = k * BSPAN
        hi = min(lo + BSPAN, N)

        def _compact(i, ptr, lo=lo, hi=hi):
            dvv = dv[pl.ds(i * 16, 16)]
            svv = sv[pl.ds(i * 16, 16)]
            m = (dvv >= lo) & (dvv < hi)
            mi = m.astype(jnp.int32)
            excl = plsc.cumsum(mi) - mi
            pos = jnp.where(m, ptr + excl, CAP + _iota16())
            plsc.store_scatter(bd, [pos], dvv - lo)
            plsc.store_scatter(bs, [pos], svv)
            return ptr + jnp.sum(mi)

        ptrs.append(lax.fori_loop(0, EPT_PAD // 16, _compact, jnp.int32(0)))
        pltpu.sync_copy(bs.at[pl.ds(0, CAP)],
                        bsrc_hbm.at[pl.ds((tid * NBUCKET + k) * CAP, CAP)])
        pltpu.sync_copy(bd.at[pl.ds(0, CAP)],
                        bdst_hbm.at[pl.ds((tid * NBUCKET + k) * CAP, CAP)])

    cvec = jnp.zeros((16,), jnp.int32)
    for k in range(NBUCKET):
        cvec = jnp.where(_iota16() == k, jnp.full((16,), ptrs[k]), cvec)
    cw[pl.ds(0, 16)] = cvec
    pltpu.sync_copy(cw, cnt16_hbm.at[pl.ds(tid * 16, 16)])


@functools.lru_cache(maxsize=None)
def _partition_call():
    return pl.kernel(
    _partition_body,
    out_type=(
        jax.ShapeDtypeStruct((NTILES * NBUCKET * CAP,), jnp.int32),
        jax.ShapeDtypeStruct((NTILES * NBUCKET * CAP,), jnp.int32),
        jax.ShapeDtypeStruct((NTILES * 16,), jnp.int32),
        jax.ShapeDtypeStruct((NCORES * NPAD,), jnp.int32),
    ),
    mesh=_mesh(),
    scratch_types=(
        pltpu.VMEM((EPT_PAD,), jnp.int32),
        pltpu.VMEM((EPT_PAD,), jnp.int32),
        pltpu.VMEM((NROWS128, 128), jnp.int32),
        pltpu.VMEM((CAP + 16,), jnp.int32),
        pltpu.VMEM((CAP + 16,), jnp.int32),
        pltpu.VMEM((16,), jnp.int32),
        pltpu.VMEM((128,), jnp.int32),
        pltpu.VMEM((640,), jnp.int32),
        pltpu.VMEM_SHARED((NPAD,), jnp.int32),
    ),
    name="gcn_edge_partition",
    compiler_params=pltpu.CompilerParams(needs_layout_passes=False),
    )


def _agg_body(u_hbm, bsrc_hbm, bdst_hbm, cnt16_hbm, z_hbm,
              s_out_hbm,
              cntv, dbuf0, dbuf1, sbuf0, sbuf1, lsrc, ldst, gidx, rows0, rows1, acc,
              sem0, sem1, lsem0, lsem1):
    c = lax.axis_index("c")
    s = lax.axis_index("s")
    tid = c * NSUB + s

    pltpu.sync_copy(cnt16_hbm, cntv.at[pl.ds(0, NTILES * 16)])

    for j in range(NSUBR // NTILES):
        r = j * NTILES + tid
        b = r // (BSPAN // SUBROWS)
        sublo = (r % (BSPAN // SUBROWS)) * SUBROWS

        pltpu.sync_copy(z_hbm, acc)

        nstrip = CAP // STRIP

        def _slot_issue(i, par, b=b):
            t = i // nstrip
            g = lax.rem(i, nstrip)
            cnt_t = cntv[pl.ds(t * 16 + b, 16)][0]

            @pl.when(g * STRIP < cnt_t)
            def _():
                base = (t * NBUCKET + b) * CAP + g * STRIP

                @pl.when(par == 0)
                def _():
                    pltpu.make_async_copy(
                        bdst_hbm.at[pl.ds(base, STRIP)], dbuf0,
                        lsem0).start()
                    pltpu.make_async_copy(
                        bsrc_hbm.at[pl.ds(base, STRIP)], sbuf0,
                        lsem0).start()

                @pl.when(par == 1)
                def _():
                    pltpu.make_async_copy(
                        bdst_hbm.at[pl.ds(base, STRIP)], dbuf1,
                        lsem1).start()
                    pltpu.make_async_copy(
                        bsrc_hbm.at[pl.ds(base, STRIP)], sbuf1,
                        lsem1).start()

        def _per_slot(i, _, b=b, sublo=sublo):
            par = lax.rem(i, 2)

            @pl.when(i + 1 < NTILES * nstrip)
            def _():
                _slot_issue(i + 1, 1 - par)

            g = lax.rem(i, nstrip)
            cnt = cntv[pl.ds((i // nstrip) * 16 + b, 16)][0]

            @pl.when(g * STRIP < cnt)
            def _():
                @pl.when(par == 0)
                def _():
                    pltpu.make_async_copy(
                        bdst_hbm.at[pl.ds(0, STRIP)], dbuf0,
                        lsem0).wait()
                    pltpu.make_async_copy(
                        bsrc_hbm.at[pl.ds(0, STRIP)], sbuf0,
                        lsem0).wait()
                    _strip_work(dbuf0, sbuf0, g, cnt, sublo)

                @pl.when(par == 1)
                def _():
                    pltpu.make_async_copy(
                        bdst_hbm.at[pl.ds(0, STRIP)], dbuf1,
                        lsem1).wait()
                    pltpu.make_async_copy(
                        bsrc_hbm.at[pl.ds(0, STRIP)], sbuf1,
                        lsem1).wait()
                    _strip_work(dbuf1, sbuf1, g, cnt, sublo)

            return 0

        def _strip_work(drow, srow, g, cnt, sublo):

                def _scan(v, lptr, g=g, cnt=cnt, sublo=sublo):
                    dl = drow[pl.ds(v * 16, 16)]
                    sv = srow[pl.ds(v * 16, 16)]
                    pos = g * STRIP + v * 16 + _iota16()
                    m = (dl >= sublo) & (dl < sublo + SUBROWS) & (pos < cnt)
                    mi = m.astype(jnp.int32)
                    incl = plsc.cumsum(mi)
                    posi = jnp.where(m, lptr + incl - mi, STRIP + _iota16())
                    plsc.store_scatter(lsrc, [posi], sv)
                    plsc.store_scatter(ldst, [posi], dl - sublo)
                    return lptr + incl[15]

                nvec = jnp.clip((cnt - g * STRIP + 15) // 16, 0, STRIP // 16)
                lptr = lax.fori_loop(0, nvec, _scan, jnp.int32(0))
                nb = (lptr + GB - 1) // GB

                def _fill(gref, bi, lptr=lptr):
                    for q in range(GB // 16):
                        pos = bi * GB + q * 16 + _iota16()
                        valid = pos < lptr
                        sv = lsrc[pl.ds(bi * GB + q * 16, 16)]
                        gref[pl.ds(q * 16, 16)] = jnp.where(
                            valid, sv, q * 16 + _iota16())

                def _issue(bi):
                    par = lax.rem(bi, 2)

                    @pl.when(par == 0)
                    def _():
                        _fill(gidx.at[0], bi)
                        pltpu.make_async_copy(
                            u_hbm.at[gidx.at[0]], rows0, sem0).start()

                    @pl.when(par == 1)
                    def _():
                        _fill(gidx.at[1], bi)
                        pltpu.make_async_copy(
                            u_hbm.at[gidx.at[1]], rows1, sem1).start()

                def _drain_acc(rows_r, bi, lptr=lptr):
                    mcount = jnp.minimum(GB, lptr - bi * GB)

                    @plsc.parallel_loop(0, mcount, unroll=2)
                    def _acc_e(e):
                        dl = ldst[pl.ds(bi * GB + e, 16)][0]
                        for cb in range(HID // 16):
                            plsc.addupdate(
                                acc.at[dl, pl.ds(cb * 16, 16)],
                                rows_r[e, pl.ds(cb * 16, 16)])

                def _gbatch(bi, _, nb=nb):
                    @pl.when(bi + 1 < nb)
                    def _():
                        _issue(bi + 1)

                    par = lax.rem(bi, 2)

                    @pl.when(par == 0)
                    def _():
                        pltpu.make_async_copy(
                            u_hbm.at[gidx.at[0]], rows0, sem0).wait()
                        _drain_acc(rows0, bi)

                    @pl.when(par == 1)
                    def _():
                        pltpu.make_async_copy(
                            u_hbm.at[gidx.at[1]], rows1, sem1).wait()
                        _drain_acc(rows1, bi)

                    return 0

                @pl.when(nb > 0)
                def _():
                    _issue(jnp.int32(0))

                lax.fori_loop(0, nb, _gbatch, 0)

        _slot_issue(jnp.int32(0), jnp.int32(0))
        lax.fori_loop(0, NTILES * nstrip, _per_slot, 0)

        pltpu.sync_copy(acc, s_out_hbm.at[pl.ds(r * SUBROWS, SUBROWS)])


@functools.lru_cache(maxsize=None)
def _agg_call():
    return pl.kernel(
    _agg_body,
    out_type=jax.ShapeDtypeStruct((NPAD, HID), jnp.float32),
    mesh=_mesh(),
    scratch_types=(
        pltpu.VMEM((NTILES * 16 + 16,), jnp.int32),
        pltpu.VMEM((STRIP,), jnp.int32),
        pltpu.VMEM((STRIP,), jnp.int32),
        pltpu.VMEM((STRIP,), jnp.int32),
        pltpu.VMEM((STRIP,), jnp.int32),
        pltpu.VMEM((STRIP + 16,), jnp.int32),
        pltpu.VMEM((STRIP + 16,), jnp.int32),
        pltpu.VMEM((2, GB), jnp.int32),
        pltpu.VMEM((GB, HID), jnp.float32),
        pltpu.VMEM((GB, HID), jnp.float32),
        pltpu.VMEM((SUBROWS, HID), jnp.float32),
        pltpu.SemaphoreType.DMA,
        pltpu.SemaphoreType.DMA,
        pltpu.SemaphoreType.DMA,
        pltpu.SemaphoreType.DMA,
    ),
    name="gcn_edge_aggregate",
    compiler_params=pltpu.CompilerParams(needs_layout_passes=False),
    )


def _mm1_body(x_ref, w_ref, c0_ref, c1_ref, u_ref, dinv_ref):
    deg = (c0_ref[...] + c1_ref[...] + 1).astype(jnp.float32)
    dinv = lax.rsqrt(deg)
    h = jnp.dot(x_ref[...], w_ref[...], preferred_element_type=jnp.float32,
                precision=_PREC)
    u_ref[...] = h * dinv
    dinv_ref[...] = dinv


def _mm1(x, w, c0, c1):
    k = x.shape[1]
    return pl.pallas_call(
        _mm1_body,
        grid=(pl.cdiv(N, RB),),
        in_specs=[
            pl.BlockSpec((RB, k), lambda i: (i, 0)),
            pl.BlockSpec((k, HID), lambda i: (0, 0)),
            pl.BlockSpec((RB, 1), lambda i: (i, 0)),
            pl.BlockSpec((RB, 1), lambda i: (i, 0)),
        ],
        out_specs=[
            pl.BlockSpec((RB, HID), lambda i: (i, 0)),
            pl.BlockSpec((RB, 1), lambda i: (i, 0)),
        ],
        out_shape=[
            jax.ShapeDtypeStruct((N, HID), jnp.float32),
            jax.ShapeDtypeStruct((N, 1), jnp.float32),
        ],
    )(x, w, c0, c1)


def _fused_body(s_ref, u_ref, dinv_ref, b_ref, w_ref, out_ref):
    dinv = dinv_ref[...]
    a = jnp.maximum((s_ref[...] + u_ref[...]) * dinv + b_ref[...], 0.0)
    out_ref[...] = jnp.dot(a, w_ref[...], preferred_element_type=jnp.float32,
                           precision=_PREC) * dinv


def _fused_mm(s_agg, u, dinv, b, w):
    return pl.pallas_call(
        _fused_body,
        grid=(pl.cdiv(N, RB),),
        in_specs=[
            pl.BlockSpec((RB, HID), lambda i: (i, 0)),
            pl.BlockSpec((RB, HID), lambda i: (i, 0)),
            pl.BlockSpec((RB, 1), lambda i: (i, 0)),
            pl.BlockSpec((1, HID), lambda i: (0, 0)),
            pl.BlockSpec((HID, HID), lambda i: (0, 0)),
        ],
        out_specs=pl.BlockSpec((RB, HID), lambda i: (i, 0)),
        out_shape=jax.ShapeDtypeStruct((N, HID), jnp.float32),
    )(s_agg, u, dinv, b, w)


def _pool_body(s_ref, u_ref, dinv_ref, b_ref, batch_ref,
               wl1_ref, bl1_ref, wl2_ref, bl2_ref, y_ref, gsum, csum):
    i = pl.program_id(0)

    @pl.when(i == 0)
    def _():
        gsum[...] = jnp.zeros_like(gsum)
        csum[...] = jnp.zeros_like(csum)

    dinv = dinv_ref[...]
    a = jnp.maximum((s_ref[...] + u_ref[...]) * dinv + b_ref[...], 0.0)
    rowpos = lax.broadcasted_iota(jnp.int32, (1, RB), 1) + i * RB
    validrow = rowpos < N
    gid = lax.broadcasted_iota(jnp.int32, (NGRAPHS, RB), 0)
    oh = jnp.where((gid == batch_ref[...]) & validrow, 1.0, 0.0)
    gsum[...] += jnp.dot(oh, a, preferred_element_type=jnp.float32,
                         precision=_PREC)
    csum[...] += jnp.sum(oh, axis=1, keepdims=True)

    @pl.when(i == pl.num_programs(0) - 1)
    def _():
        g = gsum[...] / jnp.maximum(csum[...], 1.0)
        t = jnp.maximum(
            jnp.dot(g, wl1_ref[...], preferred_element_type=jnp.float32,
                    precision=_PREC) + bl1_ref[...], 0.0)
        y_ref[...] = jnp.dot(t, wl2_ref[...],
                             preferred_element_type=jnp.float32,
                             precision=_PREC) + bl2_ref[...]


def _pool_head(s_agg, u, dinv, b, batch2d, wl1, bl1, wl2, bl2):
    return pl.pallas_call(
        _pool_body,
        grid=(pl.cdiv(N, RB),),
        in_specs=[
            pl.BlockSpec((RB, HID), lambda i: (i, 0)),
            pl.BlockSpec((RB, HID), lambda i: (i, 0)),
            pl.BlockSpec((RB, 1), lambda i: (i, 0)),
            pl.BlockSpec((1, HID), lambda i: (0, 0)),
            pl.BlockSpec((1, RB), lambda i: (0, i)),
            pl.BlockSpec((HID, HID), lambda i: (0, 0)),
            pl.BlockSpec((1, HID), lambda i: (0, 0)),
            pl.BlockSpec((HID, NCLS), lambda i: (0, 0)),
            pl.BlockSpec((1, NCLS), lambda i: (0, 0)),
        ],
        out_specs=pl.BlockSpec((NGRAPHS, NCLS), lambda i: (0, 0)),
        out_shape=jax.ShapeDtypeStruct((NGRAPHS, NCLS), jnp.float32),
        scratch_shapes=[
            pltpu.VMEM((NGRAPHS, HID), jnp.float32),
            pltpu.VMEM((NGRAPHS, 1), jnp.float32),
        ],
    )(s_agg, u, dinv, b, batch2d, wl1, bl1, wl2, bl2)


def kernel(x0, x1, edge_index0, edge_index1, batch0, batch1,
           W1_0, b1_0, W1_1, b1_1, Wc0, bc0, Wc1, bc1,
           Wl1, bl1, Wl2, bl2):
    zrows = jnp.zeros((SUBROWS, HID), jnp.float32)
    ys = []
    for (x, ei, batch, w1, b1) in ((x0, edge_index0, batch0, W1_0, b1_0),
                                   (x1, edge_index1, batch1, W1_1, b1_1)):
        src = ei[0]
        dst = ei[1]
        bsrc, bdst, cnt16, csc = _partition_call()(src, dst)
        c0 = csc[:NPAD].reshape(NPAD, 1)
        c1 = csc[NPAD:].reshape(NPAD, 1)

        u1, dinv = _mm1(x, w1, c0, c1)
        s1 = _agg_call()(u1, bsrc, bdst, cnt16, zrows)
        u2 = _fused_mm(s1, u1, dinv, b1.reshape(1, HID), Wc0)
        s2 = _agg_call()(u2, bsrc, bdst, cnt16, zrows)
        u3 = _fused_mm(s2, u2, dinv, bc0.reshape(1, HID), Wc1)
        s3 = _agg_call()(u3, bsrc, bdst, cnt16, zrows)
        y = _pool_head(s3, u3, dinv, bc1.reshape(1, HID),
                       batch.reshape(1, N), Wl1, bl1.reshape(1, HID),
                       Wl2, bl2.reshape(1, NCLS))
        ys.append(y)
    return jnp.stack(ys)
